# baseline (speedup 1.0000x reference)
import functools

import jax
import jax.numpy as jnp
from jax import lax
from jax.experimental import pallas as pl
from jax.experimental.pallas import tpu as pltpu
from jax.experimental.pallas import tpu_sc as plsc

NC = 2
NS = 16
NW = NC * NS


@functools.partial(jax.jit, static_argnames=("n", "d"))
def _gather_rows(ids_flat, table, n, d):
    rows_per_w = n // NW
    chunk = 16
    nbuf = 3
    n_chunks = rows_per_w // chunk

    mesh = plsc.VectorSubcoreMesh(core_axis_name="c", subcore_axis_name="s")

    @functools.partial(
        pl.kernel,
        out_type=jax.ShapeDtypeStruct((n, d), jnp.float32),
        mesh=mesh,
        scratch_types=[
            pltpu.VMEM((rows_per_w,), jnp.int32),
            *[pltpu.VMEM((chunk, d), jnp.float32) for _ in range(nbuf)],
            *[pltpu.SemaphoreType.DMA for _ in range(2 * nbuf)],
        ],
    )
    def k(ids_hbm, table_hbm, out_hbm, idx_v, *scr):
        bufs = scr[:nbuf]
        gsems = scr[nbuf : 2 * nbuf]
        ssems = scr[2 * nbuf :]
        wid = lax.axis_index("s") * NC + lax.axis_index("c")
        base = wid * rows_per_w
        pltpu.sync_copy(ids_hbm.at[pl.ds(base, rows_per_w)], idx_v)

        def fire_gather(g):
            p = g % nbuf
            return pltpu.async_copy(
                table_hbm.at[idx_v.at[pl.ds(g * chunk, chunk)]], bufs[p], gsems[p]
            )

        gathers = {}
        stores = {}
        for g in range(min(nbuf - 1, n_chunks)):
            gathers[g] = fire_gather(g)
        for g in range(n_chunks):
            p = g % nbuf
            gathers[g].wait()
            stores[g] = pltpu.async_copy(
                bufs[p], out_hbm.at[pl.ds(base + g * chunk, chunk)], ssems[p]
            )
            nxt = g + nbuf - 1
            if nxt < n_chunks:
                if g >= 1:
                    stores[g - 1].wait()
                gathers[nxt] = fire_gather(nxt)
        for g in range(max(0, n_chunks - nbuf), n_chunks):
            stores[g].wait()

    return k(ids_flat, table)


@functools.partial(jax.jit, static_argnames=("n", "d"))
def _tc_gather_rows(ids, table, n, d):
    def body(ids_smem, table_hbm, out_hbm, sem):
        def issue(i, carry):
            r = ids_smem[i]
            pltpu.make_async_copy(
                table_hbm.at[pl.ds(r, 1)], out_hbm.at[pl.ds(i, 1)], sem
            ).start()
            return carry

        lax.fori_loop(0, n, issue, 0)

        def drain(i, carry):
            pltpu.make_async_copy(
                table_hbm.at[pl.ds(0, 1)], out_hbm.at[pl.ds(0, 1)], sem
            ).wait()
            return carry

        lax.fori_loop(0, n, drain, 0)

    return pl.pallas_call(
        body,
        in_specs=[
            pl.BlockSpec(memory_space=pltpu.SMEM),
            pl.BlockSpec(memory_space=pl.ANY),
        ],
        out_specs=pl.BlockSpec(memory_space=pl.ANY),
        out_shape=jax.ShapeDtypeStruct((n, d), jnp.float32),
        scratch_shapes=[pltpu.SemaphoreType.DMA],
    )(ids, table)


def kernel(input_ids, table):
    b, s = input_ids.shape
    v, d = table.shape
    n = b * s
    ids_flat = input_ids.reshape(n).astype(jnp.int32)
    n_sc = (3 * n // 4) // (8 * NW) * (8 * NW)
    out_sc = _gather_rows(ids_flat[:n_sc], table, n_sc, d)
    out_tc = _tc_gather_rows(ids_flat[n_sc:], table, n - n_sc, d)
    return jnp.concatenate([out_sc, out_tc], axis=0).reshape(b, s, d)


# speedup vs baseline: 8.5487x; 8.5487x over previous
import functools

import jax
import jax.numpy as jnp
from jax import lax
from jax.experimental import pallas as pl
from jax.experimental.pallas import tpu as pltpu
from jax.experimental.pallas import tpu_sc as plsc

NC = 2
NS = 16
NW = NC * NS


@functools.partial(jax.jit, static_argnames=("n", "d"))
def _gather_rows(ids_flat, table, n, d):
    rows_per_w = n // NW
    chunk = 8
    nbuf = 6
    n_chunks = rows_per_w // chunk

    mesh = plsc.VectorSubcoreMesh(core_axis_name="c", subcore_axis_name="s")

    @functools.partial(
        pl.kernel,
        out_type=jax.ShapeDtypeStruct((n, d), jnp.float32),
        mesh=mesh,
        scratch_types=[
            pltpu.VMEM((rows_per_w,), jnp.int32),
            *[pltpu.VMEM((chunk, d), jnp.float32) for _ in range(nbuf)],
            *[pltpu.SemaphoreType.DMA for _ in range(2 * nbuf)],
        ],
    )
    def k(ids_hbm, table_hbm, out_hbm, idx_v, *scr):
        bufs = scr[:nbuf]
        gsems = scr[nbuf : 2 * nbuf]
        ssems = scr[2 * nbuf :]
        wid = lax.axis_index("s") * NC + lax.axis_index("c")
        base = wid * rows_per_w
        pltpu.sync_copy(ids_hbm.at[pl.ds(base, rows_per_w)], idx_v)

        def fire_gather(g):
            p = g % nbuf
            return pltpu.async_copy(
                table_hbm.at[idx_v.at[pl.ds(g * chunk, chunk)]], bufs[p], gsems[p]
            )

        gathers = {}
        stores = {}
        for g in range(min(nbuf - 1, n_chunks)):
            gathers[g] = fire_gather(g)
        for g in range(n_chunks):
            p = g % nbuf
            gathers[g].wait()
            stores[g] = pltpu.async_copy(
                bufs[p], out_hbm.at[pl.ds(base + g * chunk, chunk)], ssems[p]
            )
            nxt = g + nbuf - 1
            if nxt < n_chunks:
                if g >= 1:
                    stores[g - 1].wait()
                gathers[nxt] = fire_gather(nxt)
        for g in range(max(0, n_chunks - nbuf), n_chunks):
            stores[g].wait()

    return k(ids_flat, table)


def kernel(input_ids, table):
    b, s = input_ids.shape
    v, d = table.shape
    ids_flat = input_ids.reshape(b * s).astype(jnp.int32)
    out = _gather_rows(ids_flat, table, b * s, d)
    return out.reshape(b, s, d)
